# baseline (device time: 169510 ns/iter reference)
import jax
import jax.numpy as jnp
from jax import lax
from jax.experimental import pallas as pl
from jax.experimental.pallas import tpu as pltpu

Y = 4
N_HOPS = Y - 1


def kernel(x, W):
    t, d = x.shape
    _, c = W.shape
    v = Y * c

    def body(x_ref, w_ref, out_ref, send_sems, recv_sems):
        my_x = lax.axis_index("x")
        my_y = lax.axis_index("y")
        my_z = lax.axis_index("z")
        left = (my_y - 1) % Y
        right = (my_y + 1) % Y

        barrier = pltpu.get_barrier_semaphore()
        for nbr in (left, right):
            pl.semaphore_signal(
                barrier,
                inc=1,
                device_id=(my_x, nbr, my_z),
                device_id_type=pl.DeviceIdType.MESH,
            )
        pl.semaphore_wait(barrier, 2)

        e_own = jnp.exp(
            jnp.dot(x_ref[:, :], w_ref[:, :], preferred_element_type=jnp.float32)
        )
        out_ref[:, pl.ds(my_y * c, c)] = e_own
        denom = jnp.sum(e_own, axis=1, keepdims=True)

        for h in range(N_HOPS):
            src_origin = (my_y - h) % Y
            recv_origin = (my_y - h - 1) % Y
            rdma = pltpu.make_async_remote_copy(
                src_ref=out_ref.at[:, pl.ds(src_origin * c, c)],
                dst_ref=out_ref.at[:, pl.ds(src_origin * c, c)],
                send_sem=send_sems.at[h],
                recv_sem=recv_sems.at[h],
                device_id=(my_x, right, my_z),
                device_id_type=pl.DeviceIdType.MESH,
            )
            rdma.start()
            rdma.wait()
            denom = denom + jnp.sum(
                out_ref[:, pl.ds(recv_origin * c, c)], axis=1, keepdims=True
            )

        out_ref[:, :] = out_ref[:, :] * (1.0 / denom)

    return pl.pallas_call(
        body,
        out_shape=jax.ShapeDtypeStruct((t, v), jnp.float32),
        in_specs=[
            pl.BlockSpec(memory_space=pltpu.VMEM),
            pl.BlockSpec(memory_space=pltpu.VMEM),
        ],
        out_specs=pl.BlockSpec(memory_space=pltpu.VMEM),
        scratch_shapes=[
            pltpu.SemaphoreType.DMA((N_HOPS,)),
            pltpu.SemaphoreType.DMA((N_HOPS,)),
        ],
        compiler_params=pltpu.CompilerParams(collective_id=0),
    )(x, W)


# device time: 169282 ns/iter; 1.0013x vs baseline; 1.0013x over previous
import jax
import jax.numpy as jnp
from jax import lax
from jax.experimental import pallas as pl
from jax.experimental.pallas import tpu as pltpu

Y = 4
N_HOPS = Y - 1


def kernel(x, W):
    t, d = x.shape
    _, c = W.shape
    v = Y * c

    def body(x_ref, w_ref, out_ref, send_sems, recv_sems):
        my_x = lax.axis_index("x")
        my_y = lax.axis_index("y")
        my_z = lax.axis_index("z")
        left = (my_y - 1) % Y
        right = (my_y + 1) % Y

        barrier = pltpu.get_barrier_semaphore()
        for nbr in (left, right):
            pl.semaphore_signal(
                barrier,
                inc=1,
                device_id=(my_x, nbr, my_z),
                device_id_type=pl.DeviceIdType.MESH,
            )
        pl.semaphore_wait(barrier, 2)

        e_own = jnp.exp(
            jnp.dot(x_ref[:, :], w_ref[:, :], preferred_element_type=jnp.float32)
        )
        out_ref[:, pl.ds(my_y * c, c)] = e_own

        denom = jnp.zeros((t, 1), jnp.float32)
        for h in range(N_HOPS):
            src_origin = (my_y - h) % Y
            rdma = pltpu.make_async_remote_copy(
                src_ref=out_ref.at[:, pl.ds(src_origin * c, c)],
                dst_ref=out_ref.at[:, pl.ds(src_origin * c, c)],
                send_sem=send_sems.at[h],
                recv_sem=recv_sems.at[h],
                device_id=(my_x, right, my_z),
                device_id_type=pl.DeviceIdType.MESH,
            )
            rdma.start()
            denom = denom + jnp.sum(
                out_ref[:, pl.ds(src_origin * c, c)], axis=1, keepdims=True
            )
            rdma.wait()

        last_origin = (my_y + 1) % Y
        denom = denom + jnp.sum(
            out_ref[:, pl.ds(last_origin * c, c)], axis=1, keepdims=True
        )
        out_ref[:, :] = out_ref[:, :] * (1.0 / denom)

    return pl.pallas_call(
        body,
        out_shape=jax.ShapeDtypeStruct((t, v), jnp.float32),
        in_specs=[
            pl.BlockSpec(memory_space=pltpu.VMEM),
            pl.BlockSpec(memory_space=pltpu.VMEM),
        ],
        out_specs=pl.BlockSpec(memory_space=pltpu.VMEM),
        scratch_shapes=[
            pltpu.SemaphoreType.DMA((N_HOPS,)),
            pltpu.SemaphoreType.DMA((N_HOPS,)),
        ],
        compiler_params=pltpu.CompilerParams(collective_id=0),
    )(x, W)


# device time: 165042 ns/iter; 1.0271x vs baseline; 1.0257x over previous
import jax
import jax.numpy as jnp
from jax import lax
from jax.experimental import pallas as pl
from jax.experimental.pallas import tpu as pltpu

Y = 4
N_HOPS = Y - 1
S = 4


def kernel(x, W):
    t, d = x.shape
    _, c = W.shape
    v = Y * c
    sc = c // S

    def body(x_ref, w_ref, out_ref, send_sems, recv_sems):
        my_x = lax.axis_index("x")
        my_y = lax.axis_index("y")
        my_z = lax.axis_index("z")
        left = (my_y - 1) % Y
        right = (my_y + 1) % Y

        barrier = pltpu.get_barrier_semaphore()
        for nbr in (left, right):
            pl.semaphore_signal(
                barrier,
                inc=1,
                device_id=(my_x, nbr, my_z),
                device_id_type=pl.DeviceIdType.MESH,
            )
        pl.semaphore_wait(barrier, 2)

        def sub_rdma(origin, h, s):
            sl = pl.ds(origin * c + s * sc, sc)
            return pltpu.make_async_remote_copy(
                src_ref=out_ref.at[:, sl],
                dst_ref=out_ref.at[:, sl],
                send_sem=send_sems.at[h, s],
                recv_sem=recv_sems.at[h, s],
                device_id=(my_x, right, my_z),
                device_id_type=pl.DeviceIdType.MESH,
            )

        denom = jnp.zeros((t, 1), jnp.float32)
        send_rdmas = []
        for s in range(S):
            e_sub = jnp.exp(
                jnp.dot(
                    x_ref[:, :],
                    w_ref[:, pl.ds(s * sc, sc)],
                    preferred_element_type=jnp.float32,
                )
            )
            out_ref[:, pl.ds(my_y * c + s * sc, sc)] = e_sub
            r = sub_rdma(my_y, 0, s)
            r.start()
            send_rdmas.append(r)
            denom = denom + jnp.sum(e_sub, axis=1, keepdims=True)

        for h in range(1, N_HOPS):
            origin = (my_y - h) % Y
            for s in range(S):
                recv = sub_rdma(origin, h - 1, s)
                recv.wait_recv()
                fwd = sub_rdma(origin, h, s)
                fwd.start()
                send_rdmas.append(fwd)
                denom = denom + jnp.sum(
                    out_ref[:, pl.ds(origin * c + s * sc, sc)],
                    axis=1,
                    keepdims=True,
                )

        last_origin = (my_y + 1) % Y
        for s in range(S):
            recv = sub_rdma(last_origin, N_HOPS - 1, s)
            recv.wait_recv()
            denom = denom + jnp.sum(
                out_ref[:, pl.ds(last_origin * c + s * sc, sc)],
                axis=1,
                keepdims=True,
            )

        out_ref[:, :] = out_ref[:, :] * (1.0 / denom)

        for r in send_rdmas:
            r.wait_send()

    return pl.pallas_call(
        body,
        out_shape=jax.ShapeDtypeStruct((t, v), jnp.float32),
        in_specs=[
            pl.BlockSpec(memory_space=pltpu.VMEM),
            pl.BlockSpec(memory_space=pltpu.VMEM),
        ],
        out_specs=pl.BlockSpec(memory_space=pltpu.VMEM),
        scratch_shapes=[
            pltpu.SemaphoreType.DMA((N_HOPS, S)),
            pltpu.SemaphoreType.DMA((N_HOPS, S)),
        ],
        compiler_params=pltpu.CompilerParams(collective_id=0),
    )(x, W)


# device time: 111821 ns/iter; 1.5159x vs baseline; 1.4759x over previous
import jax
import jax.numpy as jnp
from jax import lax
from jax.experimental import pallas as pl
from jax.experimental.pallas import tpu as pltpu

Y = 4
N_HOPS = Y - 1
S = 2


def kernel(x, W):
    t, d = x.shape
    _, c = W.shape
    v = Y * c
    th = t // 2
    sc = c // S

    def body(x_ref, w_ref, out_ref, ysend_sems, yrecv_sems, xsend_sems, xrecv_sems):
        my_x = lax.axis_index("x")
        my_y = lax.axis_index("y")
        my_z = lax.axis_index("z")
        left = (my_y - 1) % Y
        right = (my_y + 1) % Y
        partner = 1 - my_x
        my_rows = pl.ds(my_x * th, th)

        barrier = pltpu.get_barrier_semaphore()
        for dev in ((my_x, left, my_z), (my_x, right, my_z), (partner, my_y, my_z)):
            pl.semaphore_signal(
                barrier, inc=1, device_id=dev,
                device_id_type=pl.DeviceIdType.MESH,
            )
        pl.semaphore_wait(barrier, 3)

        def piece(origin, s):
            return out_ref.at[my_rows, pl.ds(origin * c + s * sc, sc)]

        def y_rdma(origin, ev, s):
            return pltpu.make_async_remote_copy(
                src_ref=piece(origin, s), dst_ref=piece(origin, s),
                send_sem=ysend_sems.at[ev, s], recv_sem=yrecv_sems.at[ev, s],
                device_id=(my_x, right, my_z),
                device_id_type=pl.DeviceIdType.MESH,
            )

        def x_rdma(origin, ev, s):
            return pltpu.make_async_remote_copy(
                src_ref=piece(origin, s), dst_ref=piece(origin, s),
                send_sem=xsend_sems.at[ev, s], recv_sem=xrecv_sems.at[ev, s],
                device_id=(partner, my_y, my_z),
                device_id_type=pl.DeviceIdType.MESH,
            )

        d_full = jnp.zeros((t, 1), jnp.float32)
        sends = []
        for s in range(S):
            e_sub = jnp.exp(
                jnp.dot(
                    x_ref[:, :],
                    w_ref[:, pl.ds(s * sc, sc)],
                    preferred_element_type=jnp.float32,
                )
            )
            out_ref[:, pl.ds(my_y * c + s * sc, sc)] = e_sub
            r = y_rdma(my_y, 0, s)
            r.start()
            sends.append(r)
            d_full = d_full + jnp.sum(e_sub, axis=1, keepdims=True)

        d_mine = jnp.zeros((th, 1), jnp.float32)
        d_other = jnp.zeros((th, 1), jnp.float32)
        for ev in range(N_HOPS):
            origin = (my_y - ev - 1) % Y
            for s in range(S):
                y_rdma(origin, ev, s).wait_recv()
                if ev + 1 < N_HOPS:
                    fwd = y_rdma(origin, ev + 1, s)
                    fwd.start()
                    sends.append(fwd)
                xs = x_rdma(origin, ev, s)
                xs.start()
                sends.append(xs)
                d_mine = d_mine + jnp.sum(
                    out_ref[my_rows, pl.ds(origin * c + s * sc, sc)],
                    axis=1, keepdims=True,
                )

        other_rows = pl.ds(partner * th, th)
        for ev in range(N_HOPS):
            origin = (my_y - ev - 1) % Y
            for s in range(S):
                pltpu.make_async_remote_copy(
                    src_ref=piece(origin, s),
                    dst_ref=out_ref.at[other_rows, pl.ds(origin * c + s * sc, sc)],
                    send_sem=xsend_sems.at[ev, s],
                    recv_sem=xrecv_sems.at[ev, s],
                    device_id=(partner, my_y, my_z),
                    device_id_type=pl.DeviceIdType.MESH,
                ).wait_recv()
                d_other = d_other + jnp.sum(
                    out_ref[other_rows, pl.ds(origin * c + s * sc, sc)],
                    axis=1, keepdims=True,
                )

        halves = jnp.where(
            my_x == 0,
            jnp.concatenate([d_mine, d_other], axis=0),
            jnp.concatenate([d_other, d_mine], axis=0),
        )
        denom = d_full + halves
        out_ref[:, :] = out_ref[:, :] * (1.0 / denom)

        for r in sends:
            r.wait_send()

    return pl.pallas_call(
        body,
        out_shape=jax.ShapeDtypeStruct((t, v), jnp.float32),
        in_specs=[
            pl.BlockSpec(memory_space=pltpu.VMEM),
            pl.BlockSpec(memory_space=pltpu.VMEM),
        ],
        out_specs=pl.BlockSpec(memory_space=pltpu.VMEM),
        scratch_shapes=[
            pltpu.SemaphoreType.DMA((N_HOPS, S)),
            pltpu.SemaphoreType.DMA((N_HOPS, S)),
            pltpu.SemaphoreType.DMA((N_HOPS, S)),
            pltpu.SemaphoreType.DMA((N_HOPS, S)),
        ],
        compiler_params=pltpu.CompilerParams(collective_id=0),
    )(x, W)


# device time: 73063 ns/iter; 2.3201x vs baseline; 1.5305x over previous
import jax
import jax.numpy as jnp
from jax import lax
from jax.experimental import pallas as pl
from jax.experimental.pallas import tpu as pltpu

Y = 4
N_HOPS = Y - 1
S = 2


def kernel(x, W):
    t, d = x.shape
    _, c = W.shape
    v = Y * c
    th = t // 2
    sc = c // S

    def body(x_ref, w_ref, out_ref, comm_ref,
             ysend_sems, yrecv_sems, xsend_sems, xrecv_sems):
        my_x = lax.axis_index("x")
        my_y = lax.axis_index("y")
        my_z = lax.axis_index("z")
        left = (my_y - 1) % Y
        right = (my_y + 1) % Y
        partner = 1 - my_x
        my_rows = pl.ds(my_x * th, th)

        barrier = pltpu.get_barrier_semaphore()
        for dev in ((my_x, left, my_z), (my_x, right, my_z), (partner, my_y, my_z)):
            pl.semaphore_signal(
                barrier, inc=1, device_id=dev,
                device_id_type=pl.DeviceIdType.MESH,
            )
        pl.semaphore_wait(barrier, 3)

        def piece(rows, origin, s):
            return comm_ref.at[rows, pl.ds(origin * c + s * sc, sc)]

        def y_rdma(origin, ev, s):
            p = piece(my_rows, origin, s)
            return pltpu.make_async_remote_copy(
                src_ref=p, dst_ref=p,
                send_sem=ysend_sems.at[ev, s], recv_sem=yrecv_sems.at[ev, s],
                device_id=(my_x, right, my_z),
                device_id_type=pl.DeviceIdType.MESH,
            )

        def x_rdma(rows, origin, ev, s):
            p = piece(rows, origin, s)
            return pltpu.make_async_remote_copy(
                src_ref=p, dst_ref=p,
                send_sem=xsend_sems.at[ev, s], recv_sem=xrecv_sems.at[ev, s],
                device_id=(partner, my_y, my_z),
                device_id_type=pl.DeviceIdType.MESH,
            )

        d_full = jnp.zeros((t, 1), jnp.float32)
        sends = []
        for s in range(S):
            e_sub = jnp.exp(
                jnp.dot(
                    x_ref[:, :],
                    w_ref[:, pl.ds(s * sc, sc)],
                    preferred_element_type=jnp.float32,
                )
            )
            comm_ref[:, pl.ds(my_y * c + s * sc, sc)] = e_sub.astype(jnp.bfloat16)
            r = y_rdma(my_y, 0, s)
            r.start()
            sends.append(r)
            d_full = d_full + jnp.sum(e_sub, axis=1, keepdims=True)

        d_mine = jnp.zeros((th, 1), jnp.float32)
        d_other = jnp.zeros((th, 1), jnp.float32)
        for ev in range(N_HOPS):
            origin = (my_y - ev - 1) % Y
            for s in range(S):
                y_rdma(origin, ev, s).wait_recv()
                if ev + 1 < N_HOPS:
                    fwd = y_rdma(origin, ev + 1, s)
                    fwd.start()
                    sends.append(fwd)
                xs = x_rdma(my_rows, origin, ev, s)
                xs.start()
                sends.append(xs)
                d_mine = d_mine + jnp.sum(
                    comm_ref[my_rows, pl.ds(origin * c + s * sc, sc)].astype(
                        jnp.float32
                    ),
                    axis=1, keepdims=True,
                )

        other_rows = pl.ds(partner * th, th)
        for ev in range(N_HOPS):
            origin = (my_y - ev - 1) % Y
            for s in range(S):
                x_rdma(other_rows, origin, ev, s).wait_recv()
                d_other = d_other + jnp.sum(
                    comm_ref[other_rows, pl.ds(origin * c + s * sc, sc)].astype(
                        jnp.float32
                    ),
                    axis=1, keepdims=True,
                )

        halves = jnp.where(
            my_x == 0,
            jnp.concatenate([d_mine, d_other], axis=0),
            jnp.concatenate([d_other, d_mine], axis=0),
        )
        denom = d_full + halves
        inv = 1.0 / denom
        out_ref[:, :] = comm_ref[:, :].astype(jnp.float32) * inv

        for r in sends:
            r.wait_send()

    return pl.pallas_call(
        body,
        out_shape=jax.ShapeDtypeStruct((t, v), jnp.float32),
        in_specs=[
            pl.BlockSpec(memory_space=pltpu.VMEM),
            pl.BlockSpec(memory_space=pltpu.VMEM),
        ],
        out_specs=pl.BlockSpec(memory_space=pltpu.VMEM),
        scratch_shapes=[
            pltpu.VMEM((t, v), jnp.bfloat16),
            pltpu.SemaphoreType.DMA((N_HOPS, S)),
            pltpu.SemaphoreType.DMA((N_HOPS, S)),
            pltpu.SemaphoreType.DMA((N_HOPS, S)),
            pltpu.SemaphoreType.DMA((N_HOPS, S)),
        ],
        compiler_params=pltpu.CompilerParams(collective_id=0),
    )(x, W)


# device time: 73008 ns/iter; 2.3218x vs baseline; 1.0008x over previous
import jax
import jax.numpy as jnp
from jax import lax
from jax.experimental import pallas as pl
from jax.experimental.pallas import tpu as pltpu

Y = 4
N_HOPS = Y - 1
S = 2


def kernel(x, W):
    t, d = x.shape
    _, c = W.shape
    v = Y * c
    th = t // 2
    sc = c // S

    def body(x_ref, w_ref, out_ref, comm_ref,
             ysend_sems, yrecv_sems, xsend_sems, xrecv_sems):
        my_x = lax.axis_index("x")
        my_y = lax.axis_index("y")
        my_z = lax.axis_index("z")
        left = (my_y - 1) % Y
        right = (my_y + 1) % Y
        partner = 1 - my_x
        my_rows = pl.ds(my_x * th, th)

        barrier = pltpu.get_barrier_semaphore()
        for dev in ((my_x, left, my_z), (my_x, right, my_z), (partner, my_y, my_z)):
            pl.semaphore_signal(
                barrier, inc=1, device_id=dev,
                device_id_type=pl.DeviceIdType.MESH,
            )
        pl.semaphore_wait(barrier, 3)

        def piece(rows, origin, s):
            return comm_ref.at[rows, pl.ds(origin * c + s * sc, sc)]

        def y_rdma(origin, ev, s):
            p = piece(my_rows, origin, s)
            return pltpu.make_async_remote_copy(
                src_ref=p, dst_ref=p,
                send_sem=ysend_sems.at[ev, s], recv_sem=yrecv_sems.at[ev, s],
                device_id=(my_x, right, my_z),
                device_id_type=pl.DeviceIdType.MESH,
            )

        def x_rdma(rows, origin, ev, s):
            p = piece(rows, origin, s)
            return pltpu.make_async_remote_copy(
                src_ref=p, dst_ref=p,
                send_sem=xsend_sems.at[ev, s], recv_sem=xrecv_sems.at[ev, s],
                device_id=(partner, my_y, my_z),
                device_id_type=pl.DeviceIdType.MESH,
            )

        d_full = jnp.zeros((t, 1), jnp.float32)
        sends = []
        for s in range(S):
            e_sub = jnp.exp(
                jnp.dot(
                    x_ref[:, :],
                    w_ref[:, pl.ds(s * sc, sc)],
                    preferred_element_type=jnp.float32,
                )
            )
            comm_ref[:, pl.ds(my_y * c + s * sc, sc)] = e_sub.astype(jnp.bfloat16)
            r = y_rdma(my_y, 0, s)
            r.start()
            sends.append(r)
            d_full = d_full + jnp.sum(e_sub, axis=1, keepdims=True)

        d_mine = jnp.zeros((th, 1), jnp.float32)
        d_other = jnp.zeros((th, 1), jnp.float32)
        for ev in range(N_HOPS):
            origin = (my_y - ev - 1) % Y
            for s in range(S):
                y_rdma(origin, ev, s).wait_recv()
                if ev + 1 < N_HOPS:
                    fwd = y_rdma(origin, ev + 1, s)
                    fwd.start()
                    sends.append(fwd)
                xs = x_rdma(my_rows, origin, ev, s)
                xs.start()
                sends.append(xs)
                d_mine = d_mine + jnp.sum(
                    comm_ref[my_rows, pl.ds(origin * c + s * sc, sc)].astype(
                        jnp.float32
                    ),
                    axis=1, keepdims=True,
                )

        other_rows = pl.ds(partner * th, th)
        for ev in range(N_HOPS):
            origin = (my_y - ev - 1) % Y
            for s in range(S):
                x_rdma(other_rows, origin, ev, s).wait_recv()
                d_other = d_other + jnp.sum(
                    comm_ref[other_rows, pl.ds(origin * c + s * sc, sc)].astype(
                        jnp.float32
                    ),
                    axis=1, keepdims=True,
                )

        halves = jnp.where(
            my_x == 0,
            jnp.concatenate([d_mine, d_other], axis=0),
            jnp.concatenate([d_other, d_mine], axis=0),
        )
        denom = d_full + halves
        inv = 1.0 / denom
        out_ref[:, :] = comm_ref[:, :].astype(jnp.float32) * inv

        for r in sends:
            r.wait_send()

    return pl.pallas_call(
        body,
        out_shape=jax.ShapeDtypeStruct((t, v), jnp.float32),
        in_specs=[
            pl.BlockSpec(memory_space=pltpu.VMEM),
            pl.BlockSpec(memory_space=pltpu.VMEM),
        ],
        out_specs=pl.BlockSpec(memory_space=pltpu.VMEM),
        scratch_shapes=[
            pltpu.VMEM((t, v), jnp.bfloat16),
            pltpu.SemaphoreType.DMA((N_HOPS, S)),
            pltpu.SemaphoreType.DMA((N_HOPS, S)),
            pltpu.SemaphoreType.DMA((N_HOPS, S)),
            pltpu.SemaphoreType.DMA((N_HOPS, S)),
        ],
        compiler_params=pltpu.CompilerParams(collective_id=3),
    )(x, W)


# device time: 72812 ns/iter; 2.3281x vs baseline; 1.0027x over previous
import jax
import jax.numpy as jnp
from jax import lax
from jax.experimental import pallas as pl
from jax.experimental.pallas import tpu as pltpu

Y = 4
N_HOPS = Y - 1
S = 2


def kernel(x, W):
    t, d = x.shape
    _, c = W.shape
    v = Y * c
    th = t // 2
    sc = c // S

    def body(x_ref, w_ref, out_ref, comm_ref,
             ysend_sems, yrecv_sems, xsend_sems, xrecv_sems):
        my_x = lax.axis_index("x")
        my_y = lax.axis_index("y")
        my_z = lax.axis_index("z")
        left = (my_y - 1) % Y
        right = (my_y + 1) % Y
        partner = 1 - my_x
        my_rows = pl.ds(my_x * th, th)

        barrier = pltpu.get_barrier_semaphore()
        for dev in ((my_x, left, my_z), (my_x, right, my_z), (partner, my_y, my_z)):
            pl.semaphore_signal(
                barrier, inc=1, device_id=dev,
                device_id_type=pl.DeviceIdType.MESH,
            )
        pl.semaphore_wait(barrier, 3)

        def piece(rows, origin, s):
            return comm_ref.at[rows, pl.ds(origin * c + s * sc, sc)]

        def y_rdma(origin, ev, s):
            p = piece(my_rows, origin, s)
            return pltpu.make_async_remote_copy(
                src_ref=p, dst_ref=p,
                send_sem=ysend_sems.at[ev, s], recv_sem=yrecv_sems.at[ev, s],
                device_id=(my_x, right, my_z),
                device_id_type=pl.DeviceIdType.MESH,
            )

        def x_rdma(rows, origin, ev, s):
            p = piece(rows, origin, s)
            return pltpu.make_async_remote_copy(
                src_ref=p, dst_ref=p,
                send_sem=xsend_sems.at[ev, s], recv_sem=xrecv_sems.at[ev, s],
                device_id=(partner, my_y, my_z),
                device_id_type=pl.DeviceIdType.MESH,
            )

        d_full = jnp.zeros((t, 1), jnp.float32)
        sends = []
        x_bf = x_ref[:, :].astype(jnp.bfloat16)
        for s in range(S):
            e_sub = jnp.exp(
                jnp.dot(
                    x_bf,
                    w_ref[:, pl.ds(s * sc, sc)].astype(jnp.bfloat16),
                    preferred_element_type=jnp.float32,
                )
            )
            comm_ref[:, pl.ds(my_y * c + s * sc, sc)] = e_sub.astype(jnp.bfloat16)
            r = y_rdma(my_y, 0, s)
            r.start()
            sends.append(r)
            d_full = d_full + jnp.sum(e_sub, axis=1, keepdims=True)

        d_mine = jnp.zeros((th, 1), jnp.float32)
        d_other = jnp.zeros((th, 1), jnp.float32)
        for ev in range(N_HOPS):
            origin = (my_y - ev - 1) % Y
            for s in range(S):
                y_rdma(origin, ev, s).wait_recv()
                if ev + 1 < N_HOPS:
                    fwd = y_rdma(origin, ev + 1, s)
                    fwd.start()
                    sends.append(fwd)
                xs = x_rdma(my_rows, origin, ev, s)
                xs.start()
                sends.append(xs)
                d_mine = d_mine + jnp.sum(
                    comm_ref[my_rows, pl.ds(origin * c + s * sc, sc)].astype(
                        jnp.float32
                    ),
                    axis=1, keepdims=True,
                )

        other_rows = pl.ds(partner * th, th)
        for ev in range(N_HOPS):
            origin = (my_y - ev - 1) % Y
            for s in range(S):
                x_rdma(other_rows, origin, ev, s).wait_recv()
                d_other = d_other + jnp.sum(
                    comm_ref[other_rows, pl.ds(origin * c + s * sc, sc)].astype(
                        jnp.float32
                    ),
                    axis=1, keepdims=True,
                )

        halves = jnp.where(
            my_x == 0,
            jnp.concatenate([d_mine, d_other], axis=0),
            jnp.concatenate([d_other, d_mine], axis=0),
        )
        denom = d_full + halves
        inv = 1.0 / denom
        out_ref[:, :] = comm_ref[:, :].astype(jnp.float32) * inv

        for r in sends:
            r.wait_send()

    return pl.pallas_call(
        body,
        out_shape=jax.ShapeDtypeStruct((t, v), jnp.float32),
        in_specs=[
            pl.BlockSpec(memory_space=pltpu.VMEM),
            pl.BlockSpec(memory_space=pltpu.VMEM),
        ],
        out_specs=pl.BlockSpec(memory_space=pltpu.VMEM),
        scratch_shapes=[
            pltpu.VMEM((t, v), jnp.bfloat16),
            pltpu.SemaphoreType.DMA((N_HOPS, S)),
            pltpu.SemaphoreType.DMA((N_HOPS, S)),
            pltpu.SemaphoreType.DMA((N_HOPS, S)),
            pltpu.SemaphoreType.DMA((N_HOPS, S)),
        ],
        compiler_params=pltpu.CompilerParams(collective_id=3),
    )(x, W)


# device time: 63197 ns/iter; 2.6822x vs baseline; 1.1521x over previous
import jax
import jax.numpy as jnp
from jax import lax
from jax.experimental import pallas as pl
from jax.experimental.pallas import tpu as pltpu

Y = 4
N_HOPS = Y - 1
S = 2
N_BANDS = 4


def kernel(x, W):
    t, d = x.shape
    _, c = W.shape
    v = Y * c
    tb = t // N_BANDS
    sc = c // S
    hc = sc // 2

    def body(x_ref, w_ref, out_ref, comm_ref,
             y_s, y_r, xd_s, xd_r, zd_s, zd_r, xf_s, xf_r, zf_s, zf_r):
        my_x = lax.axis_index("x")
        my_y = lax.axis_index("y")
        my_z = lax.axis_index("z")
        left = (my_y - 1) % Y
        right = (my_y + 1) % Y
        zb = my_z % 2
        pz_z = my_z + 1 - 2 * zb
        ox = 1 - my_x
        r_mine = 2 * my_x + zb
        r_px = 2 * ox + zb
        r_pz = 2 * my_x + (1 - zb)
        r_d = 2 * ox + (1 - zb)

        DEV_RIGHT = (my_x, right, my_z)
        DEV_PX = (ox, my_y, my_z)
        DEV_PZ = (my_x, my_y, pz_z)

        barrier = pltpu.get_barrier_semaphore()
        for dev in ((my_x, left, my_z), DEV_RIGHT, DEV_PX, DEV_PZ):
            pl.semaphore_signal(
                barrier, inc=1, device_id=dev,
                device_id_type=pl.DeviceIdType.MESH,
            )
        pl.semaphore_wait(barrier, 4)

        def piece(band, origin, s):
            return comm_ref.at[
                pl.ds(band * tb, tb), pl.ds(origin * c + s * sc, sc)
            ]

        def half(band, origin, s, h):
            return comm_ref.at[
                pl.ds(band * tb, tb), pl.ds(origin * c + s * sc + h * hc, hc)
            ]

        def rdma(src, ss, rr, dev):
            return pltpu.make_async_remote_copy(
                src_ref=src, dst_ref=src, send_sem=ss, recv_sem=rr,
                device_id=dev, device_id_type=pl.DeviceIdType.MESH,
            )

        d_full = jnp.zeros((t, 1), jnp.float32)
        sends = []
        x_bf = x_ref[:, :].astype(jnp.bfloat16)
        for s in range(S):
            e_sub = jnp.exp(
                jnp.dot(
                    x_bf,
                    w_ref[:, pl.ds(s * sc, sc)].astype(jnp.bfloat16),
                    preferred_element_type=jnp.float32,
                )
            )
            comm_ref[:, pl.ds(my_y * c + s * sc, sc)] = e_sub.astype(jnp.bfloat16)
            rd0 = rdma(piece(r_mine, my_y, s), y_s.at[0, s], y_r.at[0, s], DEV_RIGHT)
            rd0.start()
            sends.append(rd0)
            d_full = d_full + jnp.sum(e_sub, axis=1, keepdims=True)

        def bandsum(band, origin, s, ncols):
            return jnp.sum(
                comm_ref[
                    pl.ds(band * tb, tb), pl.ds(origin * c + s * sc, ncols)
                ].astype(jnp.float32),
                axis=1, keepdims=True,
            )

        a_mine = jnp.zeros((tb, 1), jnp.float32)
        a_px = jnp.zeros((tb, 1), jnp.float32)
        a_pz = jnp.zeros((tb, 1), jnp.float32)
        a_d = jnp.zeros((tb, 1), jnp.float32)
        for ev in range(N_HOPS):
            origin = (my_y - ev - 1) % Y
            for s in range(S):
                rdma(piece(r_mine, origin, s), y_s.at[ev, s], y_r.at[ev, s],
                     DEV_RIGHT).wait_recv()
                if ev + 1 < N_HOPS:
                    f = rdma(piece(r_mine, origin, s),
                             y_s.at[ev + 1, s], y_r.at[ev + 1, s], DEV_RIGHT)
                    f.start()
                    sends.append(f)
                a = rdma(piece(r_mine, origin, s),
                         xd_s.at[ev, s], xd_r.at[ev, s], DEV_PX)
                a.start()
                sends.append(a)
                b = rdma(piece(r_mine, origin, s),
                         zd_s.at[ev, s], zd_r.at[ev, s], DEV_PZ)
                b.start()
                sends.append(b)
                a_mine = a_mine + bandsum(r_mine, origin, s, sc)

        for ev in range(N_HOPS):
            origin = (my_y - ev - 1) % Y
            for s in range(S):
                rdma(piece(r_px, origin, s), xd_s.at[ev, s], xd_r.at[ev, s],
                     DEV_PX).wait_recv()
                g = rdma(half(r_px, origin, s, zb),
                         zf_s.at[ev, s], zf_r.at[ev, s], DEV_PZ)
                g.start()
                sends.append(g)
                a_px = a_px + bandsum(r_px, origin, s, sc)

                rdma(piece(r_pz, origin, s), zd_s.at[ev, s], zd_r.at[ev, s],
                     DEV_PZ).wait_recv()
                h = rdma(half(r_pz, origin, s, zb),
                         xf_s.at[ev, s], xf_r.at[ev, s], DEV_PX)
                h.start()
                sends.append(h)
                a_pz = a_pz + bandsum(r_pz, origin, s, sc)

        for ev in range(N_HOPS):
            origin = (my_y - ev - 1) % Y
            for s in range(S):
                rdma(half(r_d, origin, s, zb), xf_s.at[ev, s], xf_r.at[ev, s],
                     DEV_PX).wait_recv()
                rdma(half(r_d, origin, s, 1 - zb), zf_s.at[ev, s],
                     zf_r.at[ev, s], DEV_PZ).wait_recv()
                a_d = a_d + bandsum(r_d, origin, s, sc)

        band_accs = []
        for b in range(N_BANDS):
            xb, zbb = b // 2, b % 2
            band_accs.append(
                jnp.where(
                    xb == my_x,
                    jnp.where(zbb == zb, a_mine, a_pz),
                    jnp.where(zbb == zb, a_px, a_d),
                )
            )
        denom = d_full + jnp.concatenate(band_accs, axis=0)
        inv = 1.0 / denom
        out_ref[:, :] = comm_ref[:, :].astype(jnp.float32) * inv

        for rr in sends:
            rr.wait_send()

    return pl.pallas_call(
        body,
        out_shape=jax.ShapeDtypeStruct((t, v), jnp.float32),
        in_specs=[
            pl.BlockSpec(memory_space=pltpu.VMEM),
            pl.BlockSpec(memory_space=pltpu.VMEM),
        ],
        out_specs=pl.BlockSpec(memory_space=pltpu.VMEM),
        scratch_shapes=[
            pltpu.VMEM((t, v), jnp.bfloat16),
            pltpu.SemaphoreType.DMA((N_HOPS, S)),
            pltpu.SemaphoreType.DMA((N_HOPS, S)),
            pltpu.SemaphoreType.DMA((N_HOPS, S)),
            pltpu.SemaphoreType.DMA((N_HOPS, S)),
            pltpu.SemaphoreType.DMA((N_HOPS, S)),
            pltpu.SemaphoreType.DMA((N_HOPS, S)),
            pltpu.SemaphoreType.DMA((N_HOPS, S)),
            pltpu.SemaphoreType.DMA((N_HOPS, S)),
            pltpu.SemaphoreType.DMA((N_HOPS, S)),
            pltpu.SemaphoreType.DMA((N_HOPS, S)),
        ],
        compiler_params=pltpu.CompilerParams(collective_id=3),
    )(x, W)
